# Initial kernel scaffold; baseline (speedup 1.0000x reference)
import jax
import jax.numpy as jnp
from jax.experimental import pallas as pl


def kernel(logprobs, beam_logprobs_sum, state, visual_key, lang_key, kv_state):
    raise NotImplementedError("write your pallas kernel here")


# speedup vs baseline: 1.7875x; 1.7875x over previous
import functools

import jax
import jax.numpy as jnp
from jax import lax
from jax.experimental import pallas as pl
from jax.experimental.pallas import tpu as pltpu

BEAM = 16
VOCAB = 100000
G = 16
W = 6272
NEG = -3e38
IMAX = 2**31 - 1


def _select_body(lp_ref, blsv_ref, bls_ref, tok_ref, topp_ref, r_ref, q_ref,
                 ybuf, gval, gidx):
    g = pl.program_id(0)
    blk = lp_ref[...]
    row = lax.broadcasted_iota(jnp.int32, (BEAM, W), 0)
    col = lax.broadcasted_iota(jnp.int32, (BEAM, W), 1) + g * W
    blk = jnp.where(col == 1, blk - 1000.0, blk)
    y = blk + blsv_ref[...]
    y = jnp.where(col >= VOCAB, NEG, y)
    ybuf[g] = y
    flat = row * VOCAB + col
    m = jnp.max(y)
    gval[g] = m
    gidx[g] = jnp.min(jnp.where(y == m, flat, IMAX))

    @pl.when(g == G - 1)
    def _():
        def round_body(t, carry):
            def scan_g(i, best):
                bv, bi, bg = best
                v = gval[i]
                fi = gidx[i]
                better = (v > bv) | ((v == bv) & (fi < bi))
                return (jnp.where(better, v, bv),
                        jnp.where(better, fi, bi),
                        jnp.where(better, i, bg))
            bv, bi, bg = lax.fori_loop(
                0, G, scan_g,
                (jnp.float32(NEG), jnp.int32(IMAX), jnp.int32(0)))
            qt = bi // VOCAB
            tok_ref[t] = bi - qt * VOCAB
            q_ref[t] = qt
            topp_ref[t] = bv
            r_ref[t] = bv - bls_ref[qt]
            yg = ybuf[bg]
            colg = lax.broadcasted_iota(jnp.int32, (BEAM, W), 1) + bg * W
            rowg = lax.broadcasted_iota(jnp.int32, (BEAM, W), 0)
            flatg = rowg * VOCAB + colg
            yg = jnp.where(flatg == bi, NEG, yg)
            ybuf[bg] = yg
            m2 = jnp.max(yg)
            gval[bg] = m2
            gidx[bg] = jnp.min(jnp.where(yg == m2, flatg, IMAX))
            return carry
        lax.fori_loop(0, BEAM, round_body, 0)


def _select(logprobs, bls):
    blsv = bls.reshape(BEAM, 1)
    out_shapes = (
        jax.ShapeDtypeStruct((BEAM,), jnp.int32),
        jax.ShapeDtypeStruct((BEAM,), jnp.float32),
        jax.ShapeDtypeStruct((BEAM,), jnp.float32),
        jax.ShapeDtypeStruct((BEAM,), jnp.int32),
    )
    smem = functools.partial(pl.BlockSpec, memory_space=pltpu.SMEM)
    return pl.pallas_call(
        _select_body,
        grid=(G,),
        in_specs=[
            pl.BlockSpec((BEAM, W), lambda g: (0, g)),
            pl.BlockSpec((BEAM, 1), lambda g: (0, 0)),
            smem(),
        ],
        out_specs=(smem(), smem(), smem(), smem()),
        out_shape=out_shapes,
        scratch_shapes=[
            pltpu.VMEM((G, BEAM, W), jnp.float32),
            pltpu.SMEM((G,), jnp.float32),
            pltpu.SMEM((G,), jnp.int32),
        ],
    )(logprobs, blsv, bls)


def _gather_body(q_ref, s_in, v_in, l_in, k_in, s_out, v_out, l_out, k_out):
    s_out[...] = s_in[...]
    v_out[...] = v_in[...]
    l_out[...] = l_in[...]
    k_out[...] = k_in[...]


def _gather(q, state, visual_key, lang_key, kv_state):
    state4 = state.reshape(2, BEAM, 1, 1024)
    grid_spec = pltpu.PrefetchScalarGridSpec(
        num_scalar_prefetch=1,
        grid=(BEAM,),
        in_specs=[
            pl.BlockSpec((2, 1, 1, 1024), lambda i, q: (0, q[i], 0, 0)),
            pl.BlockSpec((1, 196, 1024), lambda i, q: (q[i], 0, 0)),
            pl.BlockSpec((1, 512, 1024), lambda i, q: (q[i], 0, 0)),
            pl.BlockSpec((6, 1, 512, 128), lambda i, q: (0, q[i], 0, 0)),
        ],
        out_specs=[
            pl.BlockSpec((2, 1, 1, 1024), lambda i, q: (0, i, 0, 0)),
            pl.BlockSpec((1, 196, 1024), lambda i, q: (i, 0, 0)),
            pl.BlockSpec((1, 512, 1024), lambda i, q: (i, 0, 0)),
            pl.BlockSpec((6, 1, 512, 128), lambda i, q: (0, i, 0, 0)),
        ],
    )
    out_shapes = (
        jax.ShapeDtypeStruct((2, BEAM, 1, 1024), state.dtype),
        jax.ShapeDtypeStruct(visual_key.shape, visual_key.dtype),
        jax.ShapeDtypeStruct(lang_key.shape, lang_key.dtype),
        jax.ShapeDtypeStruct(kv_state.shape, kv_state.dtype),
    )
    s4, v, l, k = pl.pallas_call(
        _gather_body,
        grid_spec=grid_spec,
        out_shape=out_shapes,
    )(q, state4, visual_key, lang_key, kv_state)
    return s4.reshape(state.shape), v, l, k


def kernel(logprobs, beam_logprobs_sum, state, visual_key, lang_key, kv_state):
    tokens, top_p, r, q = _select(logprobs, beam_logprobs_sum)
    new_state, new_visual, new_lang, new_kv = _gather(
        q, state, visual_key, lang_key, kv_state)
    return (tokens, top_p, r, new_state, new_visual, new_lang, new_kv)
